# Initial kernel scaffold; baseline (speedup 1.0000x reference)
import jax
import jax.numpy as jnp
from jax.experimental import pallas as pl


def kernel(x):
    raise NotImplementedError("write your pallas kernel here")


# speedup vs baseline: 1.0025x; 1.0025x over previous
import functools

import jax
import jax.numpy as jnp
from jax import lax
from jax.experimental import pallas as pl
from jax.experimental.pallas import tpu as pltpu
from jax.experimental.pallas import tpu_sc as plsc

_F = 17
_H = 61
_W = 81
_N = _H * _W
_NPAD = 4944
_NG = _NPAD // 16
_AH = 481
_AW = 641
_R = 4
_NSTRIP = 4
_SROWS = 121
_BR = 129
_BC = 672
_NTASK = _F * _NSTRIP
_MAGIC = 12582912.0


def _sc_body(v_hbm, cx_hbm, cy_hbm, sc_hbm, acc_hbm, v_v, cx_v, cy_v, sc_v,
             wl_v, buf):
    wid = lax.axis_index("s") * 2 + lax.axis_index("c")
    lanes = lax.iota(jnp.int32, 16)
    zero16f = jnp.zeros((16,), jnp.float32)
    zero16i = jnp.zeros((16,), jnp.int32)

    for t in range(3):
        i = wid + 32 * t
        live = i < _NTASK

        @pl.when(live)
        def _task(i=i):
            s = lax.shift_right_logical(i * 3856, 16)
            f = i - 17 * s
            y_lo = jnp.where(s == 0, -8, 121 * s)
            y_hi = jnp.where(s == 3, 488, 121 * s + 120)

            pltpu.sync_copy(v_hbm.at[f], v_v)
            pltpu.sync_copy(cx_hbm.at[f], cx_v)
            pltpu.sync_copy(cy_hbm.at[f], cy_v)
            pltpu.sync_copy(sc_hbm.at[f], sc_v)

            def zero_row(r, c):
                for k in range(_BC // 16):
                    buf[r, pl.ds(16 * k, 16)] = zero16f
                return c
            lax.fori_loop(0, _BR, zero_row, 0)

            def scan_g(g, cnt):
                base = g * 16
                v = v_v[pl.ds(base, 16)]
                cyy = cy_v[pl.ds(base, 16)] * 8.0
                scc = sc_v[pl.ds(base, 16)] * 8.0
                valid = (v > 0.1) & (scc >= 0.0)
                ry = jnp.clip(cyy, -100.0, 584.0)
                ry = (ry + _MAGIC) - _MAGIC
                cy0 = jnp.clip(ry, -4.0, 484.0).astype(jnp.int32)
                hit = valid & (cy0 + _R >= y_lo) & (cy0 - _R <= y_hi)
                pos = cnt + plsc.cumsum(hit.astype(jnp.int32)) - 1
                plsc.store_scatter(wl_v, [pos], base + lanes, mask=hit)
                return cnt + plsc.all_reduce_population_count(hit)
            cnt = lax.fori_loop(0, _NG, scan_g, zero16i)

            def cond(j):
                return jnp.any(j < cnt)

            def point(j):
                pidx = plsc.load_gather(wl_v, [j])
                v = plsc.load_gather(v_v, [pidx])
                cxx = plsc.load_gather(cx_v, [pidx]) * 8.0
                cyy = plsc.load_gather(cy_v, [pidx]) * 8.0
                scc = plsc.load_gather(sc_v, [pidx]) * 8.0
                sig = jnp.maximum(1.0, 0.5 * scc)
                sig2 = sig * sig
                val = v * 0.0625
                rx = jnp.clip(cxx, -100.0, 744.0)
                rx = (rx + _MAGIC) - _MAGIC
                cx0 = jnp.clip(rx, -4.0, 644.0).astype(jnp.int32)
                ry = jnp.clip(cyy, -100.0, 584.0)
                ry = (ry + _MAGIC) - _MAGIC
                cy0 = jnp.clip(ry, -4.0, 484.0).astype(jnp.int32)
                for u in range(6):
                    k = u * 16 + lanes
                    dyq = lax.shift_right_logical(k * 57, 9)
                    dy = dyq - 4
                    dx = k - 9 * dyq - 4
                    xx = cx0 + dx
                    yy = cy0 + dy
                    xxc = jnp.clip(xx, 0, _AW - 1)
                    yyc = jnp.clip(yy, 0, _AH - 1)
                    fdx = xxc.astype(jnp.float32) - cxx
                    fdy = yyc.astype(jnp.float32) - cyy
                    dx2 = fdx * fdx
                    dy2 = fdy * fdy
                    d2 = dx2 + dy2
                    nearest = (dx2 < 0.25) & (dy2 < 0.25)
                    w = jnp.where(nearest, val,
                                  val * jnp.exp((-0.5 * d2) / sig2))
                    m = (d2 <= sig2) & (k < 81) & (yy >= y_lo) & (yy <= y_hi)
                    row = jnp.where(yy < 0, yy + 129, yy - 121 * s)
                    row = jnp.clip(row, 0, _BR - 1)
                    col = jnp.where(xx < 0, xx + 657, xx)
                    plsc.addupdate_scatter(buf, [row, col], w, mask=m)
                return j + 1
            lax.while_loop(cond, point, zero16i)

            any_pts = jnp.any(cnt > 0)

            @pl.when(any_pts & (s == 0))
            def _fold_top():
                for k in range(_BC // 16):
                    acc_v = buf[0, pl.ds(16 * k, 16)]
                    for h in range(121, 129):
                        acc_v = acc_v + buf[h, pl.ds(16 * k, 16)]
                    buf[0, pl.ds(16 * k, 16)] = acc_v

            @pl.when(any_pts & (s == 3))
            def _fold_bottom():
                for k in range(_BC // 16):
                    acc_v = buf[117, pl.ds(16 * k, 16)]
                    for h in range(118, 126):
                        acc_v = acc_v + buf[h, pl.ds(16 * k, 16)]
                    buf[117, pl.ds(16 * k, 16)] = acc_v

            @pl.when(any_pts)
            def _fold_x_and_clamp():
                def fold_rows(rr, c):
                    rows = rr * 16 + lanes
                    left = plsc.load_gather(buf, [rows, zero16i])
                    for cc in range(649, 657):
                        left = left + plsc.load_gather(buf, [rows, cc + zero16i])
                    plsc.store_scatter(buf, [rows, zero16i], left)
                    right = plsc.load_gather(buf, [rows, 640 + zero16i])
                    for cc in range(641, 649):
                        right = right + plsc.load_gather(buf, [rows, cc + zero16i])
                    plsc.store_scatter(buf, [rows, 640 + zero16i], right)
                    return c
                lax.fori_loop(0, 8, fold_rows, 0)

                def clamp_row(r, c):
                    for k in range(41):
                        q = buf[r, pl.ds(16 * k, 16)]
                        buf[r, pl.ds(16 * k, 16)] = jnp.minimum(q, 1.0)
                    return c
                lax.fori_loop(0, 121, clamp_row, 0)

            @pl.when(s < 3)
            def _out_main():
                pltpu.sync_copy(buf.at[pl.ds(0, 121), :],
                                acc_hbm.at[f, pl.ds(121 * s, 121), :])

            @pl.when(s == 3)
            def _out_last():
                pltpu.sync_copy(buf.at[pl.ds(0, 118), :],
                                acc_hbm.at[f, pl.ds(363, 118), :])


_accumulate_call = pl.kernel(
    _sc_body,
    out_type=jax.ShapeDtypeStruct((_F, _AH, _BC), jnp.float32),
    mesh=plsc.VectorSubcoreMesh(core_axis_name="c", subcore_axis_name="s",
                                num_cores=2, num_subcores=16),
    scratch_types=[
        pltpu.VMEM((_NPAD,), jnp.float32),
        pltpu.VMEM((_NPAD,), jnp.float32),
        pltpu.VMEM((_NPAD,), jnp.float32),
        pltpu.VMEM((_NPAD,), jnp.float32),
        pltpu.VMEM((_NPAD,), jnp.int32),
        pltpu.VMEM((_BR, _BC), jnp.float32),
    ],
)


def _accumulate(x):
    chans = x[1:].reshape(_F, 5, _N)
    pad = ((0, 0), (0, _NPAD - _N))
    v = jnp.pad(chans[:, 0], pad)
    cx = jnp.pad(chans[:, 1], pad)
    cy = jnp.pad(chans[:, 2], pad)
    sc = jnp.pad(chans[:, 4], pad)
    return _accumulate_call(v, cx, cy, sc)


def kernel(x):
    acc = _accumulate(x)
    out, _ = lax.optimization_barrier((x, acc))
    return out
